# Initial kernel scaffold; baseline (speedup 1.0000x reference)
import jax
import jax.numpy as jnp
from jax.experimental import pallas as pl


def kernel(hidden_states, router_input, router_scale, router_proj, per_expert_scale, w1, w2, w3):
    raise NotImplementedError("write your pallas kernel here")


# speedup vs baseline: 2.3298x; 2.3298x over previous
import functools

import jax
import jax.numpy as jnp
from jax.experimental import pallas as pl
from jax.experimental.pallas import tpu as pltpu

HIDDEN = 768
NUM_EXPERTS = 8
TOP_K = 2
DFF = 1024
TOKENS = 2048
EPS = 1e-06


def _router_body(rin_ref, rscale_ref, rproj_ref, pes_ref, comb_ref):
    x = rin_ref[...]
    var = jnp.mean(jnp.square(x), axis=-1, keepdims=True)
    x = x * jax.lax.rsqrt(var + EPS)
    x = x * rscale_ref[...] * (HIDDEN ** -0.5)
    logits = jnp.dot(
        x.astype(jnp.bfloat16),
        rproj_ref[...].astype(jnp.bfloat16),
        preferred_element_type=jnp.float32,
    )
    probs = jax.nn.softmax(logits, axis=-1)

    iota = jax.lax.broadcasted_iota(jnp.int32, probs.shape, 1)
    m1 = jnp.max(probs, axis=-1, keepdims=True)
    a1 = jnp.min(jnp.where(probs == m1, iota, NUM_EXPERTS), axis=-1, keepdims=True)
    one1 = (iota == a1)
    probs2 = jnp.where(one1, -jnp.inf, probs)
    m2 = jnp.max(probs2, axis=-1, keepdims=True)
    a2 = jnp.min(jnp.where(probs2 == m2, iota, NUM_EXPERTS), axis=-1, keepdims=True)
    one2 = (iota == a2)

    denom = m1 + m2 + 1e-20
    comb = (m1 * one1 + m2 * one2) / denom
    comb_ref[...] = comb * pes_ref[...]


def _expert_body(x_ref, comb_ref, w1_ref, w3_ref, w2_ref, out_ref):
    e = pl.program_id(0)
    x = x_ref[...].astype(jnp.bfloat16)
    g = jnp.dot(x, w1_ref[0].astype(jnp.bfloat16), preferred_element_type=jnp.float32)
    u = jnp.dot(x, w3_ref[0].astype(jnp.bfloat16), preferred_element_type=jnp.float32)
    h = jax.nn.gelu(g) * u
    y = jnp.dot(h.astype(jnp.bfloat16), w2_ref[0].astype(jnp.bfloat16),
                preferred_element_type=jnp.float32)
    lane = jax.lax.broadcasted_iota(jnp.int32, (1, NUM_EXPERTS), 1)
    c = jnp.sum(comb_ref[...] * (lane == e).astype(jnp.float32), axis=-1,
                keepdims=True)
    contrib = c * y

    @pl.when(e == 0)
    def _():
        out_ref[...] = contrib

    @pl.when(e != 0)
    def _():
        out_ref[...] += contrib


@jax.jit
def kernel(hidden_states, router_input, router_scale, router_proj,
           per_expert_scale, w1, w2, w3):
    T, H = hidden_states.shape
    E = NUM_EXPERTS

    combine = pl.pallas_call(
        _router_body,
        out_shape=jax.ShapeDtypeStruct((T, E), jnp.float32),
        in_specs=[
            pl.BlockSpec((T, H), lambda: (0, 0)),
            pl.BlockSpec((1, H), lambda: (0, 0)),
            pl.BlockSpec((H, E), lambda: (0, 0)),
            pl.BlockSpec((1, E), lambda: (0, 0)),
        ],
        out_specs=pl.BlockSpec((T, E), lambda: (0, 0)),
    )(router_input, router_scale.reshape(1, H), router_proj,
      per_expert_scale.reshape(1, E))

    out = pl.pallas_call(
        _expert_body,
        grid=(E,),
        out_shape=jax.ShapeDtypeStruct((T, H), jnp.float32),
        in_specs=[
            pl.BlockSpec((T, H), lambda e: (0, 0)),
            pl.BlockSpec((T, E), lambda e: (0, 0)),
            pl.BlockSpec((1, H, DFF), lambda e: (e, 0, 0)),
            pl.BlockSpec((1, H, DFF), lambda e: (e, 0, 0)),
            pl.BlockSpec((1, DFF, H), lambda e: (e, 0, 0)),
        ],
        out_specs=pl.BlockSpec((T, H), lambda e: (0, 0)),
        compiler_params=pltpu.CompilerParams(
            dimension_semantics=("arbitrary",),
        ),
    )(hidden_states, combine, w1, w3, w2)
    return out
